# Initial kernel scaffold; baseline (speedup 1.0000x reference)
import jax
import jax.numpy as jnp
from jax.experimental import pallas as pl


def kernel(points, features, npoint):
    raise NotImplementedError("write your pallas kernel here")


# speedup vs baseline: 27.3945x; 27.3945x over previous
import jax
import jax.numpy as jnp
from jax import lax
from jax.experimental import pallas as pl
from jax.experimental.pallas import tpu as pltpu

_NPOINT = 512


def _fps_kernel(pts_ref, out_ref, dist_ref):
    _, B, N = pts_ref.shape
    px = pts_ref[0]
    py = pts_ref[1]
    pz = pts_ref[2]
    lane = lax.broadcasted_iota(jnp.int32, (B, N), 1)
    ocol = lax.broadcasted_iota(jnp.int32, (B, _NPOINT), 1)

    dist_ref[...] = jnp.full((B, N), 1e10, dtype=jnp.float32)
    out_ref[...] = jnp.zeros((B, _NPOINT), dtype=jnp.int32)

    def body(i, far):
        out_ref[...] = jnp.where(ocol == i, far, out_ref[...])
        sel = lane == far
        cx = jnp.sum(jnp.where(sel, px, 0.0), axis=1, keepdims=True)
        cy = jnp.sum(jnp.where(sel, py, 0.0), axis=1, keepdims=True)
        cz = jnp.sum(jnp.where(sel, pz, 0.0), axis=1, keepdims=True)
        d = (px - cx) ** 2 + (py - cy) ** 2 + (pz - cz) ** 2
        dist = jnp.minimum(dist_ref[...], d)
        dist_ref[...] = dist
        mx = jnp.max(dist, axis=1, keepdims=True)
        far_new = jnp.min(jnp.where(dist == mx, lane, N), axis=1, keepdims=True)
        return far_new

    far0 = jnp.zeros((B, 1), dtype=jnp.int32)
    lax.fori_loop(0, _NPOINT, body, far0)


def kernel(points, features, npoint):
    del features, npoint
    B, N, _ = points.shape
    pts_t = jnp.transpose(points, (2, 0, 1))
    out = pl.pallas_call(
        _fps_kernel,
        out_shape=jax.ShapeDtypeStruct((B, _NPOINT), jnp.int32),
        scratch_shapes=[pltpu.VMEM((B, N), jnp.float32)],
    )(pts_t)
    return out
